# Initial kernel scaffold; baseline (speedup 1.0000x reference)
import jax
import jax.numpy as jnp
from jax.experimental import pallas as pl


def kernel(X0_patches, H0_patches, Wq, Wk):
    raise NotImplementedError("write your pallas kernel here")


# speedup vs baseline: 1.1980x; 1.1980x over previous
import jax
import jax.numpy as jnp
import numpy as np
from jax.experimental import pallas as pl

_B, _N, _D, _QK, _KEEP = 4, 2048, 1024, 128, 256
_SCALE = np.float32(_QK ** (-0.5))


def _proj_kernel(x_ref, w_ref, o_ref):
    o_ref[0] = jnp.dot(x_ref[0], w_ref[...],
                       preferred_element_type=jnp.float32).astype(jnp.bfloat16)


def _proj(x, w):
    return pl.pallas_call(
        _proj_kernel,
        grid=(_B,),
        in_specs=[pl.BlockSpec((1, _N, _D), lambda b: (b, 0, 0)),
                  pl.BlockSpec((_D, _QK), lambda b: (0, 0))],
        out_specs=pl.BlockSpec((1, _N, _QK), lambda b: (b, 0, 0)),
        out_shape=jax.ShapeDtypeStruct((_B, _N, _QK), jnp.bfloat16),
    )(x, w)


def _slse_kernel(q_ref, k_ref, o_ref):
    st = jax.lax.dot_general(k_ref[0], q_ref[0], (((1,), (1,)), ((), ())),
                             preferred_element_type=jnp.float32)
    ss = st * _SCALE
    m = jnp.max(ss, axis=0)
    e = jnp.exp(ss - m[None, :])
    o_ref[0, 0, :] = jnp.log(jnp.abs(jnp.sum(e, axis=0))) + m


def _scores(qb, kb):
    blk = 512
    return pl.pallas_call(
        _slse_kernel,
        grid=(_B, _N // blk),
        in_specs=[pl.BlockSpec((1, blk, _QK), lambda b, i: (b, i, 0)),
                  pl.BlockSpec((1, _N, _QK), lambda b, i: (b, 0, 0))],
        out_specs=pl.BlockSpec((1, 1, blk), lambda b, i: (b, 0, i)),
        out_shape=jax.ShapeDtypeStruct((_B, 1, _N), jnp.float32),
    )(qb, kb)


def _topk_gather_kernel(s_ref, h_ref, idx_ref, out_ref):
    s = s_ref[0, 0, :]
    s_row = s[None, :]
    chunk = 512
    ranks = []
    for c in range(_N // chunk):
        s_i = s[c * chunk:(c + 1) * chunk][:, None]
        i_ids = (jax.lax.broadcasted_iota(jnp.int32, (chunk, _N), 0)
                 + c * chunk)
        j_ids = jax.lax.broadcasted_iota(jnp.int32, (chunk, _N), 1)
        gt = (s_row > s_i)
        eq_lt = (s_row == s_i) & (j_ids < i_ids)
        cnt = jnp.where(gt | eq_lt, 1, 0).sum(axis=1)
        ranks.append(cnt)
    rank = jnp.concatenate(ranks, axis=0)
    k_ids = jax.lax.broadcasted_iota(jnp.int32, (_N, _KEEP), 1)
    onehot = jnp.where(rank[:, None] == k_ids, 1.0, 0.0)
    i_col = jax.lax.broadcasted_iota(jnp.int32, (_N, _KEEP), 0)
    idx_ref[0, 0, :] = jnp.sum(jnp.where(rank[:, None] == k_ids, i_col, 0),
                               axis=0)
    out_ref[0] = jax.lax.dot_general(
        onehot, h_ref[0], (((0,), (0,)), ((), ())),
        preferred_element_type=jnp.float32)


def _topk_gather(scores3, h0):
    return pl.pallas_call(
        _topk_gather_kernel,
        grid=(_B,),
        in_specs=[pl.BlockSpec((1, 1, _N), lambda b: (b, 0, 0)),
                  pl.BlockSpec((1, _N, _D), lambda b: (b, 0, 0))],
        out_specs=[pl.BlockSpec((1, 1, _KEEP), lambda b: (b, 0, 0)),
                   pl.BlockSpec((1, _KEEP, _D), lambda b: (b, 0, 0))],
        out_shape=[jax.ShapeDtypeStruct((_B, 1, _KEEP), jnp.int32),
                   jax.ShapeDtypeStruct((_B, _KEEP, _D), jnp.float32)],
    )(scores3, h0)


def kernel(X0_patches, H0_patches, Wq, Wk):
    kb = _proj(X0_patches, Wk)
    qb = _proj(H0_patches, Wq)
    scores3 = _scores(qb, kb)
    idx3, out = _topk_gather(scores3, H0_patches)
    return (out, idx3[:, 0, :], scores3[:, 0, :])
